# baseline (speedup 1.0000x reference)
import functools

import jax
import jax.numpy as jnp
from jax import lax
from jax.experimental import pallas as pl
from jax.experimental.pallas import tpu as pltpu
from jax.experimental.pallas import tpu_sc as plsc

HOT_NUMS = 50000
HASH_SIZE = 450000
EMBED_DIM = 64
LANES = 16
IDX_CHUNK = 128
ROUND = 256


@functools.cache
def _build(B, D):
    info = plsc.get_sparse_core_info()
    NC, NS = info.num_cores, info.num_subcores
    NW = NC * NS
    bpw = B // NW
    n_rounds = bpw // ROUND
    n_chunks = ROUND // IDX_CHUNK
    mesh = plsc.VectorSubcoreMesh(core_axis_name="c", subcore_axis_name="s")

    @functools.partial(
        pl.kernel,
        mesh=mesh,
        out_type=jax.ShapeDtypeStruct((B // 2, 2 * D), jnp.float32),
        compiler_params=pltpu.CompilerParams(use_tc_tiling_on_sc=False),
        scratch_types=[
            pltpu.VMEM((bpw,), jnp.int32),
            pltpu.VMEM((n_chunks, IDX_CHUNK), jnp.int32),
            pltpu.VMEM((n_chunks, IDX_CHUNK), jnp.int32),
            pltpu.VMEM((ROUND, 2 * D), jnp.float32),
            pltpu.VMEM((ROUND, 2 * D), jnp.float32),
            pltpu.SemaphoreType.DMA,
            pltpu.SemaphoreType.DMA,
        ],
    )
    def sc_kernel(in_hbm, whp_hbm, whashp_hbm, out_hbm,
                  ids_v, hot_v, cold_v, hp, cp, sem_h, sem_c):
        wid = lax.axis_index("s") * NC + lax.axis_index("c")
        base = wid * bpw
        pltpu.sync_copy(in_hbm.at[pl.ds(base, bpw)], ids_v)

        for r in range(n_rounds):
            r_off = r * ROUND

            def idx_body(g, carry, r_off=r_off):
                v = ids_v[pl.ds(r_off + g * LANES, LANES)]
                d = jnp.abs(v)
                chunk = g // (IDX_CHUNK // LANES)
                off = lax.rem(g * LANES, IDX_CHUNK)
                hot_v[chunk, pl.ds(off, LANES)] = lax.shift_right_logical(lax.rem(d, HOT_NUMS), 1)
                cold_v[chunk, pl.ds(off, LANES)] = lax.shift_right_logical(lax.rem(d, HASH_SIZE), 1)
                return carry

            lax.fori_loop(0, ROUND // LANES, idx_body, 0)

            for k in range(n_chunks):
                pltpu.async_copy(
                    whp_hbm.at[hot_v.at[k]],
                    hp.at[pl.ds(k * IDX_CHUNK, IDX_CHUNK)], sem_h)
                pltpu.async_copy(
                    whashp_hbm.at[cold_v.at[k]],
                    cp.at[pl.ds(k * IDX_CHUNK, IDX_CHUNK)], sem_c)
            pltpu.make_async_copy(
                whp_hbm.at[pl.ds(0, ROUND)], hp, sem_h).wait()
            pltpu.make_async_copy(
                whashp_hbm.at[pl.ds(0, ROUND)], cp, sem_c).wait()

            def sel_body(g, carry, r_off=r_off):
                v = ids_v[pl.ds(r_off + g * LANES, LANES)]
                d = jnp.abs(v)
                hot_vec = lax.rem(v, 10) == 0
                ph = lax.rem(lax.rem(d, HOT_NUMS), 2)
                pc = lax.rem(lax.rem(d, HASH_SIZE), 2)
                code = jnp.where(hot_vec, 2 + ph, pc)
                for l in range(LANES):
                    j = g * LANES + l
                    cd = code[l]
                    dst = (l % 2) * D

                    def move(src_ref, src_half, j=j, dst=dst):
                        for c in range(D // LANES):
                            src_ref_row = src_ref[
                                j, pl.ds(src_half + c * LANES, LANES)]
                            cp[j // 2, pl.ds(dst + c * LANES, LANES)] = (
                                src_ref_row)

                    @pl.when(cd == 0)
                    def _():
                        move(cp, 0)

                    @pl.when(cd == 1)
                    def _():
                        move(cp, D)

                    @pl.when(cd == 2)
                    def _():
                        move(hp, 0)

                    @pl.when(cd == 3)
                    def _():
                        move(hp, D)
                return carry

            lax.fori_loop(0, ROUND // LANES, sel_body, 0)
            pltpu.sync_copy(
                cp.at[pl.ds(0, ROUND // 2)],
                out_hbm.at[pl.ds((base + r_off) // 2, ROUND // 2)])

    return sc_kernel


def kernel(input, offsets, weight_h, weight_hash):
    del offsets
    B = input.shape[0]
    whp = weight_h.reshape(HOT_NUMS // 2, 2 * EMBED_DIM)
    whashp = weight_hash.reshape(HASH_SIZE // 2, 2 * EMBED_DIM)
    out_p = _build(B, EMBED_DIM)(input, whp, whashp)
    return out_p.reshape(B, EMBED_DIM)


# speedup vs baseline: 1.0611x; 1.0611x over previous
import functools

import jax
import jax.numpy as jnp
from jax import lax
from jax.experimental import pallas as pl
from jax.experimental.pallas import tpu as pltpu
from jax.experimental.pallas import tpu_sc as plsc

HOT_NUMS = 50000
HASH_SIZE = 450000
EMBED_DIM = 64
LANES = 16
IDX_CHUNK = 128


@functools.cache
def _build(B, D):
    info = plsc.get_sparse_core_info()
    NC, NS = info.num_cores, info.num_subcores
    NW = NC * NS
    bpw = B // NW
    n_chunks = bpw // IDX_CHUNK
    mesh = plsc.VectorSubcoreMesh(core_axis_name="c", subcore_axis_name="s")

    @functools.partial(
        pl.kernel,
        mesh=mesh,
        out_type=jax.ShapeDtypeStruct((B, D), jnp.float32),
        compiler_params=pltpu.CompilerParams(use_tc_tiling_on_sc=False),
        scratch_types=[
            pltpu.VMEM((bpw,), jnp.int32),
            pltpu.VMEM((n_chunks, IDX_CHUNK), jnp.int32),
            pltpu.VMEM((n_chunks, IDX_CHUNK), jnp.int32),
            pltpu.VMEM((bpw, D), jnp.float32),
            pltpu.VMEM((bpw, D), jnp.float32),
            pltpu.SemaphoreType.DMA,
            pltpu.SemaphoreType.DMA,
        ],
    )
    def sc_kernel(in_hbm, wh_hbm, whash_hbm, out_hbm,
                  ids_v, hot_v, cold_v, hrows, crows, sem_h, sem_c):
        wid = lax.axis_index("s") * NC + lax.axis_index("c")
        base = wid * bpw
        pltpu.sync_copy(in_hbm.at[pl.ds(base, bpw)], ids_v)

        def idx_body(g, carry):
            v = ids_v[pl.ds(g * LANES, LANES)]
            d = jnp.abs(v)
            chunk = g // (IDX_CHUNK // LANES)
            off = lax.rem(g * LANES, IDX_CHUNK)
            hot_v[chunk, pl.ds(off, LANES)] = lax.rem(d, HOT_NUMS)
            cold_v[chunk, pl.ds(off, LANES)] = lax.rem(d, HASH_SIZE)
            return carry

        lax.fori_loop(0, bpw // LANES, idx_body, 0)

        for k in range(n_chunks):
            pltpu.async_copy(
                wh_hbm.at[hot_v.at[k]],
                hrows.at[pl.ds(k * IDX_CHUNK, IDX_CHUNK)], sem_h)
            pltpu.async_copy(
                whash_hbm.at[cold_v.at[k]],
                crows.at[pl.ds(k * IDX_CHUNK, IDX_CHUNK)], sem_c)
        pltpu.make_async_copy(wh_hbm.at[pl.ds(0, bpw)], hrows, sem_h).wait()
        pltpu.make_async_copy(whash_hbm.at[pl.ds(0, bpw)], crows, sem_c).wait()

        def sel_body(g, carry):
            m = lax.rem(ids_v[pl.ds(g * LANES, LANES)], 10)
            for l in range(LANES):
                j = g * LANES + l
                hot = m[l] == 0
                for c in range(D // LANES):
                    h = hrows[j, pl.ds(c * LANES, LANES)]
                    cl = crows[j, pl.ds(c * LANES, LANES)]
                    crows[j, pl.ds(c * LANES, LANES)] = jnp.where(hot, h, cl)
            return carry

        lax.fori_loop(0, bpw // LANES, sel_body, 0)
        pltpu.sync_copy(crows, out_hbm.at[pl.ds(base, bpw)])

    return sc_kernel


def kernel(input, offsets, weight_h, weight_hash):
    del offsets
    B = input.shape[0]
    return _build(B, EMBED_DIM)(input, weight_h, weight_hash)
